# baseline (speedup 1.0000x reference)
import numpy as np
import jax
import jax.numpy as jnp
from jax import lax
from jax.experimental import pallas as pl
from jax.experimental.pallas import tpu as pltpu
from jax.experimental.pallas import tpu_sc as plsc

NC = 2
NS = 16
NW = NC * NS
L = 16

CHUNK_Q = 2
STEP_CHUNKS = 2
SIGMA = 0.7
_RSQRT_MAGIC = 0x5F3759DF


def _newton_rsqrt(x):
    i = plsc.bitcast(x, jnp.int32)
    i = _RSQRT_MAGIC - lax.shift_right_logical(i, 1)
    y = plsc.bitcast(i, jnp.float32)
    for _ in range(3):
        y = y * (1.5 - 0.5 * x * y * y)
    return y


def _argmin_tree(scores):
    items = [(s, jnp.full((L,), i, jnp.int32)) for i, s in enumerate(scores)]
    while len(items) > 1:
        nxt = []
        for a in range(0, len(items) - 1, 2):
            (sa, ia), (sb, ib) = items[a], items[a + 1]
            lt = sb < sa
            nxt.append((jnp.where(lt, sb, sa), jnp.where(lt, ib, ia)))
        if len(items) % 2:
            nxt.append(items[-1])
        items = nxt
    return items[0]


def _make_body(n_pad, m, m_pad, h, c, k):
    q_per_w = n_pad // NW
    chunks = q_per_w // CHUNK_Q
    rows = CHUNK_Q * h
    cd = c // 2
    nj = cd // L

    def body(sx_hbm, sy_hbm, sz_hbm, qx_hbm, qy_hbm, qz_hbm, inds_hbm,
             w_hbm, kp_hbm, tab_hbm, out_hbm,
             sx_v, sy_v, sz_v, qx_v, qy_v, qz_v, inds_v, ib2_0, ib2_1,
             w_v, kp_v, sf_sh, buf0, buf1, cf_v, ki_v, out_v,
             sem0, sem1):
        sid = lax.axis_index("s")
        wid = sid * NC + lax.axis_index("c")
        base = wid * q_per_w

        m_share = m_pad // 2 // NS
        for i in range(m_share // rows):
            off = sid * m_share + i * rows
            pltpu.sync_copy(tab_hbm.at[pl.ds(off, rows)], buf0)
            pltpu.sync_copy(buf0, sf_sh.at[pl.ds(off, rows)])

        pltpu.sync_copy(sx_hbm, sx_v)
        pltpu.sync_copy(sy_hbm, sy_v)
        pltpu.sync_copy(sz_hbm, sz_v)
        pltpu.sync_copy(qx_hbm.at[pl.ds(base, q_per_w)],
                        qx_v.at[pl.ds(0, q_per_w)])
        pltpu.sync_copy(qy_hbm.at[pl.ds(base, q_per_w)],
                        qy_v.at[pl.ds(0, q_per_w)])
        pltpu.sync_copy(qz_hbm.at[pl.ds(base, q_per_w)],
                        qz_v.at[pl.ds(0, q_per_w)])
        pltpu.sync_copy(inds_hbm.at[pl.ds(base * h, q_per_w * h)], inds_v)
        pltpu.sync_copy(w_hbm, w_v)
        pltpu.sync_copy(kp_hbm, kp_v)

        kx2 = [kp_v[0][i] for i in range(k)]
        ky2 = [kp_v[1][i] for i in range(k)]
        kz2 = [kp_v[2][i] for i in range(k)]
        ckk = [kp_v[3][i] for i in range(k)]
        plsc.subcore_barrier()

        ib2s = (ib2_0, ib2_1)

        def start(buf, sem, p, chunk):
            for t in range(rows // L):
                iv = inds_v[pl.ds(chunk * rows + t * L, L)]
                ib2s[p][pl.ds(t * L, L)] = lax.shift_right_logical(iv, 1)
            pltpu.make_async_copy(sf_sh.at[ib2s[p]], buf, sem).start()

        def wait(buf, sem, p):
            pltpu.make_async_copy(sf_sh.at[ib2s[p]], buf, sem).wait()

        def dist(chunk):
            for q in range(CHUNK_Q):
                ql = chunk * CHUNK_Q + q
                qx = qx_v[pl.ds(ql, L)][0]
                qy = qy_v[pl.ds(ql, L)][0]
                qz = qz_v[pl.ds(ql, L)][0]
                for v in range(h // L):
                    idx16 = inds_v[pl.ds(chunk * rows + q * h + v * L, L)]
                    rx = plsc.load_gather(sx_v, [idx16]) - qx
                    ry = plsc.load_gather(sy_v, [idx16]) - qy
                    rz = plsc.load_gather(sz_v, [idx16]) - qz
                    rr = rx * rx + ry * ry + rz * rz
                    scores = [
                        ckk[kk] + rx * kx2[kk] + ry * ky2[kk] + rz * kz2[kk]
                        for kk in range(k)]
                    best, bidx = _argmin_tree(scores)
                    d2 = jnp.maximum(rr + best, 0.0) + 1e-30
                    sq = d2 * _newton_rsqrt(d2)
                    coeff = jnp.maximum(1.0 - sq * (1.0 / SIGMA), 0.0)
                    cf_v[pl.ds(q * h + v * L, L)] = coeff
                    ki_v[pl.ds(q * h + v * L, L)] = bidx | ((idx16 & 1) << 4)

        hi_mask = jnp.int32(-65536)

        def aggregate(buf, off):
            for q in range(CHUNK_Q):
                unroll = 4

                def agg(t, acc):
                    e0 = q * h + t * unroll
                    cfv = cf_v[pl.ds(e0, L)]
                    kiv = ki_v[pl.ds(e0, L)]
                    acc = list(acc)
                    for u in range(unroll):
                        csc = cfv[u]
                        kv = kiv[u]
                        kk = kv & 15
                        hb = (kv >> 4) << 6
                        for j in range(nj):
                            vj = buf[e0 + u, pl.ds(hb + j * L, L)]
                            fe = plsc.bitcast(vj << 16, jnp.float32)
                            fo = plsc.bitcast(vj & hi_mask, jnp.float32)
                            we = w_v[kk, pl.ds(j * 2 * L, L)]
                            wo = w_v[kk, pl.ds((j * 2 + 1) * L, L)]
                            acc[2 * j] = acc[2 * j] + (csc * we) * fe
                            acc[2 * j + 1] = acc[2 * j + 1] + (csc * wo) * fo
                    return tuple(acc)

                acc = lax.fori_loop(
                    0, h // unroll, agg,
                    tuple(jnp.zeros((L,), jnp.float32) for _ in range(2 * nj)))
                for j in range(2 * nj):
                    out_v[off + q, pl.ds(j * L, L)] = acc[j]

        bufs = (buf0, buf1)
        sems = (sem0, sem1)
        start(buf0, sem0, 0, 0)
        start(buf1, sem1, 1, 1)

        def step(g, carry):
            for s in range(STEP_CHUNKS):
                p = s % 2
                cs = STEP_CHUNKS * g + s
                dist(cs)
                wait(bufs[p], sems[p], p)
                aggregate(bufs[p], s * CHUNK_Q)
                nxt = jnp.minimum(cs + 2, chunks - 2 + p)
                start(bufs[p], sems[p], p, nxt)
            pltpu.sync_copy(
                out_v,
                out_hbm.at[pl.ds(base + g * STEP_CHUNKS * CHUNK_Q,
                                 STEP_CHUNKS * CHUNK_Q)])
            return carry

        lax.fori_loop(0, chunks // STEP_CHUNKS, step, 0)
        wait(buf0, sem0, 0)
        wait(buf1, sem1, 1)

    return body, q_per_w, rows


def kernel(q_pts, s_pts, s_feats, neighb_inds, weights, kernel_points):
    n, _ = q_pts.shape
    m, c = s_feats.shape
    h = neighb_inds.shape[1]
    k = kernel_points.shape[0]

    n_pad = ((n + NW * 8 - 1) // (NW * 8)) * (NW * 8)
    rows_per_chunk = CHUNK_Q * h
    m_unit = 2 * NS * rows_per_chunk
    m_pad = ((m + m_unit - 1) // m_unit) * m_unit
    body, q_per_w, rows = _make_body(n_pad, m, m_pad, h, c, k)

    fb = s_feats.astype(jnp.bfloat16)
    fu = lax.bitcast_convert_type(fb, jnp.uint16)
    lo = fu[:, 0::2].astype(jnp.uint32)
    hi = fu[:, 1::2].astype(jnp.uint32)
    tab = lax.bitcast_convert_type(lo | (hi << 16), jnp.int32)
    tab = jnp.pad(tab, ((0, m_pad - m), (0, 0)))
    tab = tab.reshape(m_pad // 2, c)

    sx = s_pts[:, 0]
    sy = s_pts[:, 1]
    sz = s_pts[:, 2]
    qp = jnp.pad(q_pts, ((0, n_pad - n), (0, 0)))
    qx = qp[:, 0]
    qy = qp[:, 1]
    qz = qp[:, 2]
    inds = jnp.pad(neighb_inds, ((0, n_pad - n), (0, 0))).reshape(-1)

    perm = []
    for j in range(c // 32):
        perm.extend(2 * (j * L + l) for l in range(L))
        perm.extend(2 * (j * L + l) + 1 for l in range(L))
    perm = np.asarray(perm)
    inv_perm = np.argsort(perm)
    w_pad = jnp.pad(weights, ((0, 16 - k), (0, 0)))[:, perm]

    kp2 = -2.0 * kernel_points
    ck = jnp.sum(kernel_points * kernel_points, axis=1)
    kp_pack = jnp.stack([
        jnp.pad(kp2[:, 0], (0, 16 - k)),
        jnp.pad(kp2[:, 1], (0, 16 - k)),
        jnp.pad(kp2[:, 2], (0, 16 - k)),
        jnp.pad(ck, (0, 16 - k)),
    ])

    mesh = plsc.VectorSubcoreMesh(core_axis_name="c", subcore_axis_name="s",
                                  num_cores=NC, num_subcores=NS)
    run = pl.kernel(
        body,
        out_type=jax.ShapeDtypeStruct((n_pad, c), jnp.float32),
        mesh=mesh,
        compiler_params=pltpu.CompilerParams(needs_layout_passes=False),
        scratch_types=[
            pltpu.VMEM((m,), jnp.float32),
            pltpu.VMEM((m,), jnp.float32),
            pltpu.VMEM((m,), jnp.float32),
            pltpu.VMEM((q_per_w + L,), jnp.float32),
            pltpu.VMEM((q_per_w + L,), jnp.float32),
            pltpu.VMEM((q_per_w + L,), jnp.float32),
            pltpu.VMEM((q_per_w * h,), jnp.int32),
            pltpu.VMEM((rows,), jnp.int32),
            pltpu.VMEM((rows,), jnp.int32),
            pltpu.VMEM((16, c), jnp.float32),
            pltpu.VMEM((4, 16), jnp.float32),
            pltpu.VMEM_SHARED((m_pad // 2, c), jnp.int32),
            pltpu.VMEM((rows, c), jnp.int32),
            pltpu.VMEM((rows, c), jnp.int32),
            pltpu.VMEM((rows + L,), jnp.float32),
            pltpu.VMEM((rows + L,), jnp.int32),
            pltpu.VMEM((STEP_CHUNKS * CHUNK_Q, c), jnp.float32),
            pltpu.SemaphoreType.DMA,
            pltpu.SemaphoreType.DMA,
        ],
    )
    out = run(sx, sy, sz, qx, qy, qz, inds, w_pad, kp_pack, tab)
    return out[:n][:, inv_perm]


# speedup vs baseline: 1.0075x; 1.0075x over previous
import numpy as np
import jax
import jax.numpy as jnp
from jax import lax
from jax.experimental import pallas as pl
from jax.experimental.pallas import tpu as pltpu
from jax.experimental.pallas import tpu_sc as plsc

NC = 2
NS = 16
NW = NC * NS
L = 16

CHUNK_Q = 2
STEP_CHUNKS = 2
SIGMA = 0.7
_RSQRT_MAGIC = 0x5F3759DF


def _newton_rsqrt(x):
    i = plsc.bitcast(x, jnp.int32)
    i = _RSQRT_MAGIC - lax.shift_right_logical(i, 1)
    y = plsc.bitcast(i, jnp.float32)
    for _ in range(2):
        y = y * (1.5 - 0.5 * x * y * y)
    return y


def _argmin_tree(scores):
    items = [(s, jnp.full((L,), i, jnp.int32)) for i, s in enumerate(scores)]
    while len(items) > 1:
        nxt = []
        for a in range(0, len(items) - 1, 2):
            (sa, ia), (sb, ib) = items[a], items[a + 1]
            lt = sb < sa
            nxt.append((jnp.where(lt, sb, sa), jnp.where(lt, ib, ia)))
        if len(items) % 2:
            nxt.append(items[-1])
        items = nxt
    return items[0]


def _make_body(n_pad, m, m_pad, h, c, k):
    q_per_w = n_pad // NW
    chunks = q_per_w // CHUNK_Q
    rows = CHUNK_Q * h
    cd = c // 2
    nj = cd // L

    def body(sx_hbm, sy_hbm, sz_hbm, qx_hbm, qy_hbm, qz_hbm, inds_hbm,
             w_hbm, kp_hbm, tab_hbm, out_hbm,
             sx_v, sy_v, sz_v, qx_v, qy_v, qz_v, inds_v, ib2_0, ib2_1,
             w_v, kp_v, sf_sh, buf0, buf1, cf_v, ki_v, out_v,
             sem0, sem1):
        sid = lax.axis_index("s")
        wid = sid * NC + lax.axis_index("c")
        base = wid * q_per_w

        m_share = m_pad // 2 // NS
        for i in range(m_share // rows):
            off = sid * m_share + i * rows
            pltpu.sync_copy(tab_hbm.at[pl.ds(off, rows)], buf0)
            pltpu.sync_copy(buf0, sf_sh.at[pl.ds(off, rows)])

        pltpu.sync_copy(sx_hbm, sx_v)
        pltpu.sync_copy(sy_hbm, sy_v)
        pltpu.sync_copy(sz_hbm, sz_v)
        pltpu.sync_copy(qx_hbm.at[pl.ds(base, q_per_w)],
                        qx_v.at[pl.ds(0, q_per_w)])
        pltpu.sync_copy(qy_hbm.at[pl.ds(base, q_per_w)],
                        qy_v.at[pl.ds(0, q_per_w)])
        pltpu.sync_copy(qz_hbm.at[pl.ds(base, q_per_w)],
                        qz_v.at[pl.ds(0, q_per_w)])
        pltpu.sync_copy(inds_hbm.at[pl.ds(base * h, q_per_w * h)], inds_v)
        pltpu.sync_copy(w_hbm, w_v)
        pltpu.sync_copy(kp_hbm, kp_v)

        kx2 = [kp_v[0][i] for i in range(k)]
        ky2 = [kp_v[1][i] for i in range(k)]
        kz2 = [kp_v[2][i] for i in range(k)]
        ckk = [kp_v[3][i] for i in range(k)]
        plsc.subcore_barrier()

        ib2s = (ib2_0, ib2_1)

        def start(buf, sem, p, chunk):
            for t in range(rows // L):
                iv = inds_v[pl.ds(chunk * rows + t * L, L)]
                ib2s[p][pl.ds(t * L, L)] = lax.shift_right_logical(iv, 1)
            pltpu.make_async_copy(sf_sh.at[ib2s[p]], buf, sem).start()

        def wait(buf, sem, p):
            pltpu.make_async_copy(sf_sh.at[ib2s[p]], buf, sem).wait()

        def dist(chunk):
            for q in range(CHUNK_Q):
                ql = chunk * CHUNK_Q + q
                qx = qx_v[pl.ds(ql, L)][0]
                qy = qy_v[pl.ds(ql, L)][0]
                qz = qz_v[pl.ds(ql, L)][0]
                for v in range(h // L):
                    idx16 = inds_v[pl.ds(chunk * rows + q * h + v * L, L)]
                    rx = plsc.load_gather(sx_v, [idx16]) - qx
                    ry = plsc.load_gather(sy_v, [idx16]) - qy
                    rz = plsc.load_gather(sz_v, [idx16]) - qz
                    rr = rx * rx + ry * ry + rz * rz
                    scores = [
                        ckk[kk] + rx * kx2[kk] + ry * ky2[kk] + rz * kz2[kk]
                        for kk in range(k)]
                    best, bidx = _argmin_tree(scores)
                    d2 = jnp.maximum(rr + best, 0.0) + 1e-30
                    sq = d2 * _newton_rsqrt(d2)
                    coeff = jnp.maximum(1.0 - sq * (1.0 / SIGMA), 0.0)
                    cf_v[pl.ds(q * h + v * L, L)] = coeff
                    ki_v[pl.ds(q * h + v * L, L)] = bidx | ((idx16 & 1) << 4)

        hi_mask = jnp.int32(-65536)

        def aggregate(buf, off):
            for q in range(CHUNK_Q):
                unroll = 4

                def agg(t, acc):
                    e0 = q * h + t * unroll
                    cfv = cf_v[pl.ds(e0, L)]
                    kiv = ki_v[pl.ds(e0, L)]
                    acc = list(acc)
                    for u in range(unroll):
                        csc = cfv[u]
                        kv = kiv[u]
                        kk = kv & 15
                        hb = (kv >> 4) << 6
                        for j in range(nj):
                            vj = buf[e0 + u, pl.ds(hb + j * L, L)]
                            fe = plsc.bitcast(vj << 16, jnp.float32)
                            fo = plsc.bitcast(vj & hi_mask, jnp.float32)
                            we = w_v[kk, pl.ds(j * 2 * L, L)]
                            wo = w_v[kk, pl.ds((j * 2 + 1) * L, L)]
                            acc[2 * j] = acc[2 * j] + (csc * we) * fe
                            acc[2 * j + 1] = acc[2 * j + 1] + (csc * wo) * fo
                    return tuple(acc)

                acc = lax.fori_loop(
                    0, h // unroll, agg,
                    tuple(jnp.zeros((L,), jnp.float32) for _ in range(2 * nj)))
                for j in range(2 * nj):
                    out_v[off + q, pl.ds(j * L, L)] = acc[j]

        bufs = (buf0, buf1)
        sems = (sem0, sem1)
        start(buf0, sem0, 0, 0)
        start(buf1, sem1, 1, 1)

        def step(g, carry):
            for s in range(STEP_CHUNKS):
                p = s % 2
                cs = STEP_CHUNKS * g + s
                dist(cs)
                wait(bufs[p], sems[p], p)
                aggregate(bufs[p], s * CHUNK_Q)
                nxt = jnp.minimum(cs + 2, chunks - 2 + p)
                start(bufs[p], sems[p], p, nxt)
            pltpu.sync_copy(
                out_v,
                out_hbm.at[pl.ds(base + g * STEP_CHUNKS * CHUNK_Q,
                                 STEP_CHUNKS * CHUNK_Q)])
            return carry

        lax.fori_loop(0, chunks // STEP_CHUNKS, step, 0)
        wait(buf0, sem0, 0)
        wait(buf1, sem1, 1)

    return body, q_per_w, rows


def kernel(q_pts, s_pts, s_feats, neighb_inds, weights, kernel_points):
    n, _ = q_pts.shape
    m, c = s_feats.shape
    h = neighb_inds.shape[1]
    k = kernel_points.shape[0]

    n_pad = ((n + NW * 8 - 1) // (NW * 8)) * (NW * 8)
    rows_per_chunk = CHUNK_Q * h
    m_unit = 2 * NS * rows_per_chunk
    m_pad = ((m + m_unit - 1) // m_unit) * m_unit
    body, q_per_w, rows = _make_body(n_pad, m, m_pad, h, c, k)

    fb = s_feats.astype(jnp.bfloat16)
    fu = lax.bitcast_convert_type(fb, jnp.uint16)
    lo = fu[:, 0::2].astype(jnp.uint32)
    hi = fu[:, 1::2].astype(jnp.uint32)
    tab = lax.bitcast_convert_type(lo | (hi << 16), jnp.int32)
    tab = jnp.pad(tab, ((0, m_pad - m), (0, 0)))
    tab = tab.reshape(m_pad // 2, c)

    sx = s_pts[:, 0]
    sy = s_pts[:, 1]
    sz = s_pts[:, 2]
    qp = jnp.pad(q_pts, ((0, n_pad - n), (0, 0)))
    qx = qp[:, 0]
    qy = qp[:, 1]
    qz = qp[:, 2]
    inds = jnp.pad(neighb_inds, ((0, n_pad - n), (0, 0))).reshape(-1)

    perm = []
    for j in range(c // 32):
        perm.extend(2 * (j * L + l) for l in range(L))
        perm.extend(2 * (j * L + l) + 1 for l in range(L))
    perm = np.asarray(perm)
    inv_perm = np.argsort(perm)
    w_pad = jnp.pad(weights, ((0, 16 - k), (0, 0)))[:, perm]

    kp2 = -2.0 * kernel_points
    ck = jnp.sum(kernel_points * kernel_points, axis=1)
    kp_pack = jnp.stack([
        jnp.pad(kp2[:, 0], (0, 16 - k)),
        jnp.pad(kp2[:, 1], (0, 16 - k)),
        jnp.pad(kp2[:, 2], (0, 16 - k)),
        jnp.pad(ck, (0, 16 - k)),
    ])

    mesh = plsc.VectorSubcoreMesh(core_axis_name="c", subcore_axis_name="s",
                                  num_cores=NC, num_subcores=NS)
    run = pl.kernel(
        body,
        out_type=jax.ShapeDtypeStruct((n_pad, c), jnp.float32),
        mesh=mesh,
        compiler_params=pltpu.CompilerParams(needs_layout_passes=False),
        scratch_types=[
            pltpu.VMEM((m,), jnp.float32),
            pltpu.VMEM((m,), jnp.float32),
            pltpu.VMEM((m,), jnp.float32),
            pltpu.VMEM((q_per_w + L,), jnp.float32),
            pltpu.VMEM((q_per_w + L,), jnp.float32),
            pltpu.VMEM((q_per_w + L,), jnp.float32),
            pltpu.VMEM((q_per_w * h,), jnp.int32),
            pltpu.VMEM((rows,), jnp.int32),
            pltpu.VMEM((rows,), jnp.int32),
            pltpu.VMEM((16, c), jnp.float32),
            pltpu.VMEM((4, 16), jnp.float32),
            pltpu.VMEM_SHARED((m_pad // 2, c), jnp.int32),
            pltpu.VMEM((rows, c), jnp.int32),
            pltpu.VMEM((rows, c), jnp.int32),
            pltpu.VMEM((rows + L,), jnp.float32),
            pltpu.VMEM((rows + L,), jnp.int32),
            pltpu.VMEM((STEP_CHUNKS * CHUNK_Q, c), jnp.float32),
            pltpu.SemaphoreType.DMA,
            pltpu.SemaphoreType.DMA,
        ],
    )
    out = run(sx, sy, sz, qx, qy, qz, inds, w_pad, kp_pack, tab)
    return out[:n][:, inv_perm]
